# baseline (device time: 29192 ns/iter reference)
import jax
import jax.numpy as jnp
from jax import lax
from jax.experimental import pallas as pl
from jax.experimental.pallas import tpu as pltpu

N_DEV = 4


def kernel(x):
    _, m, n = x.shape
    half = m // 2
    qtr = m // 4

    def body(x_ref, out_ref, xv, stg_a, stg_b, recv_a1, recv_b1, recv_a2,
             recv_b2, acc_a, acc_b, csem, ssem, rsem):
        p = lax.axis_index("i")
        q = p ^ 1
        r = 3 - p

        j = jnp.where((p == 1) | (p == 2), 1, 0)
        jb = p // 2

        rows = [(1 - j) * qtr, half + (1 - jb) * qtr, j * qtr, half + jb * qtr]
        copies = []
        for i, row in enumerate(rows):
            cp = pltpu.make_async_copy(
                x_ref.at[0, pl.ds(row, qtr), :], xv.at[i], csem.at[i])
            cp.start()
            copies.append(cp)

        barrier_sem = pltpu.get_barrier_semaphore()
        for nbr in (q, r):
            pl.semaphore_signal(
                barrier_sem, inc=1,
                device_id=(nbr,), device_id_type=pl.DeviceIdType.MESH,
            )
        pl.semaphore_wait(barrier_sem, 2)

        def rdma(src, dst, sem_idx, dev):
            return pltpu.make_async_remote_copy(
                src_ref=src, dst_ref=dst,
                send_sem=ssem.at[sem_idx], recv_sem=rsem.at[sem_idx],
                device_id=(dev,), device_id_type=pl.DeviceIdType.MESH,
            )

        copies[0].wait()
        stg_a[...] = xv[0].astype(jnp.bfloat16)
        a1 = rdma(stg_a, recv_a1, 0, q)
        a1.start()
        copies[1].wait()
        stg_b[...] = xv[1].astype(jnp.bfloat16)
        b1 = rdma(stg_b, recv_b1, 1, r)
        b1.start()

        a1.wait()
        copies[2].wait()
        acc_a[...] = xv[2].astype(jnp.bfloat16) + recv_a1[...]
        a2 = rdma(acc_a, recv_a2, 2, r)
        a2.start()

        b1.wait()
        copies[3].wait()
        acc_b[...] = xv[3].astype(jnp.bfloat16) + recv_b1[...]
        b2 = rdma(acc_b, recv_b2, 3, q)
        b2.start()

        a2.wait()
        out_ref[pl.ds(j * qtr, qtr), :] = acc_a[...] + recv_a2[...]
        a3 = rdma(out_ref.at[pl.ds(j * qtr, qtr)],
                  out_ref.at[pl.ds(j * qtr, qtr)], 4, q)
        a3.start()

        b2.wait()
        out_ref[pl.ds(half + jb * qtr, qtr), :] = acc_b[...] + recv_b2[...]
        b3 = rdma(out_ref.at[pl.ds(half + jb * qtr, qtr)],
                  out_ref.at[pl.ds(half + jb * qtr, qtr)], 5, r)
        b3.start()

        a3.wait()
        b3.wait()

    return pl.pallas_call(
        body,
        out_shape=jax.ShapeDtypeStruct((m, n), jnp.bfloat16),
        in_specs=[pl.BlockSpec(memory_space=pl.ANY)],
        out_specs=pl.BlockSpec(memory_space=pltpu.VMEM),
        scratch_shapes=[
            pltpu.VMEM((4, qtr, n), jnp.float32),
            pltpu.VMEM((qtr, n), jnp.bfloat16),
            pltpu.VMEM((qtr, n), jnp.bfloat16),
            pltpu.VMEM((qtr, n), jnp.bfloat16),
            pltpu.VMEM((qtr, n), jnp.bfloat16),
            pltpu.VMEM((qtr, n), jnp.bfloat16),
            pltpu.VMEM((qtr, n), jnp.bfloat16),
            pltpu.VMEM((qtr, n), jnp.bfloat16),
            pltpu.VMEM((qtr, n), jnp.bfloat16),
            pltpu.SemaphoreType.DMA((4,)),
            pltpu.SemaphoreType.DMA((6,)),
            pltpu.SemaphoreType.DMA((6,)),
        ],
        compiler_params=pltpu.CompilerParams(collective_id=0),
    )(x)


# device time: 25397 ns/iter; 1.1494x vs baseline; 1.1494x over previous
import jax
import jax.numpy as jnp
from jax import lax
from jax.experimental import pallas as pl
from jax.experimental.pallas import tpu as pltpu

N_DEV = 4


def kernel(x):
    _, m, n = x.shape
    half = m // 2
    qtr = m // 4
    sub = m // 8

    def body(x_ref, out_ref, stg_a, stg_b, recv_a1, recv_b1, recv_a2,
             recv_b2, acc_a, acc_b, ssem, rsem):
        p = lax.axis_index("i")
        q = p ^ 1
        r = 3 - p

        barrier_sem = pltpu.get_barrier_semaphore()
        for nbr in (q, r):
            pl.semaphore_signal(
                barrier_sem, inc=1,
                device_id=(nbr,), device_id_type=pl.DeviceIdType.MESH,
            )
        pl.semaphore_wait(barrier_sem, 2)

        j = jnp.where((p == 1) | (p == 2), 1, 0)
        jb = p // 2

        def rdma(src, dst, sem_idx, dev):
            return pltpu.make_async_remote_copy(
                src_ref=src, dst_ref=dst,
                send_sem=ssem.at[sem_idx], recv_sem=rsem.at[sem_idx],
                device_id=(dev,), device_id_type=pl.DeviceIdType.MESH,
            )

        def xbf16(row_start):
            return x_ref[0, pl.ds(row_start, sub), :].astype(jnp.bfloat16)

        ds = pl.ds

        a1 = [None, None]
        b1 = [None, None]
        for s in range(2):
            stg_a[ds(s * sub, sub), :] = xbf16((1 - j) * qtr + s * sub)
            a1[s] = rdma(stg_a.at[ds(s * sub, sub)],
                         recv_a1.at[ds(s * sub, sub)], 0 + s, q)
            a1[s].start()
            stg_b[ds(s * sub, sub), :] = xbf16(half + (1 - jb) * qtr + s * sub)
            b1[s] = rdma(stg_b.at[ds(s * sub, sub)],
                         recv_b1.at[ds(s * sub, sub)], 2 + s, r)
            b1[s].start()

        a2 = [None, None]
        b2 = [None, None]
        for s in range(2):
            a1[s].wait()
            acc_a[ds(s * sub, sub), :] = (
                xbf16(j * qtr + s * sub) + recv_a1[ds(s * sub, sub), :])
            a2[s] = rdma(acc_a.at[ds(s * sub, sub)],
                         recv_a2.at[ds(s * sub, sub)], 4 + s, r)
            a2[s].start()
            b1[s].wait()
            acc_b[ds(s * sub, sub), :] = (
                xbf16(half + jb * qtr + s * sub) + recv_b1[ds(s * sub, sub), :])
            b2[s] = rdma(acc_b.at[ds(s * sub, sub)],
                         recv_b2.at[ds(s * sub, sub)], 6 + s, q)
            b2[s].start()

        a3 = [None, None]
        b3 = [None, None]
        for s in range(2):
            a2[s].wait()
            out_ref[ds(j * qtr + s * sub, sub), :] = (
                acc_a[ds(s * sub, sub), :] + recv_a2[ds(s * sub, sub), :])
            a3[s] = rdma(out_ref.at[ds(j * qtr + s * sub, sub)],
                         out_ref.at[ds(j * qtr + s * sub, sub)], 8 + s, q)
            a3[s].start()
            b2[s].wait()
            out_ref[ds(half + jb * qtr + s * sub, sub), :] = (
                acc_b[ds(s * sub, sub), :] + recv_b2[ds(s * sub, sub), :])
            b3[s] = rdma(out_ref.at[ds(half + jb * qtr + s * sub, sub)],
                         out_ref.at[ds(half + jb * qtr + s * sub, sub)],
                         10 + s, r)
            b3[s].start()

        for s in range(2):
            a3[s].wait()
            b3[s].wait()

    return pl.pallas_call(
        body,
        out_shape=jax.ShapeDtypeStruct((m, n), jnp.bfloat16),
        in_specs=[pl.BlockSpec(memory_space=pltpu.VMEM)],
        out_specs=pl.BlockSpec(memory_space=pltpu.VMEM),
        scratch_shapes=[
            pltpu.VMEM((qtr, n), jnp.bfloat16),
            pltpu.VMEM((qtr, n), jnp.bfloat16),
            pltpu.VMEM((qtr, n), jnp.bfloat16),
            pltpu.VMEM((qtr, n), jnp.bfloat16),
            pltpu.VMEM((qtr, n), jnp.bfloat16),
            pltpu.VMEM((qtr, n), jnp.bfloat16),
            pltpu.VMEM((qtr, n), jnp.bfloat16),
            pltpu.VMEM((qtr, n), jnp.bfloat16),
            pltpu.SemaphoreType.DMA((12,)),
            pltpu.SemaphoreType.DMA((12,)),
        ],
        compiler_params=pltpu.CompilerParams(collective_id=0),
    )(x)


# device time: 25358 ns/iter; 1.1512x vs baseline; 1.0015x over previous
import jax
import jax.numpy as jnp
from jax import lax
from jax.experimental import pallas as pl
from jax.experimental.pallas import tpu as pltpu

N_DEV = 4


def kernel(x):
    _, m, n = x.shape
    half = m // 2
    qtr = m // 4
    sub = m // 8

    def body(x_ref, out_ref, stg_a, stg_b, keep_a, keep_b, recv_a1, recv_b1,
             recv_a2, recv_b2, acc_a, acc_b, ssem, rsem):
        p = lax.axis_index("i")
        q = p ^ 1
        r = 3 - p

        j = jnp.where((p == 1) | (p == 2), 1, 0)
        jb = p // 2

        def rdma(src, dst, sem_idx, dev):
            return pltpu.make_async_remote_copy(
                src_ref=src, dst_ref=dst,
                send_sem=ssem.at[sem_idx], recv_sem=rsem.at[sem_idx],
                device_id=(dev,), device_id_type=pl.DeviceIdType.MESH,
            )

        def xqtr(row_start):
            return x_ref[0, pl.ds(row_start, qtr), :].astype(jnp.bfloat16)

        ds = pl.ds

        barrier_sem = pltpu.get_barrier_semaphore()
        for nbr in (q, r):
            pl.semaphore_signal(
                barrier_sem, inc=1,
                device_id=(nbr,), device_id_type=pl.DeviceIdType.MESH,
            )
        stg_a[...] = xqtr((1 - j) * qtr)
        stg_b[...] = xqtr(half + (1 - jb) * qtr)
        keep_a[...] = xqtr(j * qtr)
        keep_b[...] = xqtr(half + jb * qtr)
        pl.semaphore_wait(barrier_sem, 2)

        a1 = [None, None]
        b1 = [None, None]
        for s in range(2):
            a1[s] = rdma(stg_a.at[ds(s * sub, sub)],
                         recv_a1.at[ds(s * sub, sub)], 0 + s, q)
            a1[s].start()
            b1[s] = rdma(stg_b.at[ds(s * sub, sub)],
                         recv_b1.at[ds(s * sub, sub)], 2 + s, r)
            b1[s].start()

        a2 = [None, None]
        b2 = [None, None]
        for s in range(2):
            a1[s].wait()
            acc_a[ds(s * sub, sub), :] = (
                keep_a[ds(s * sub, sub), :] + recv_a1[ds(s * sub, sub), :])
            a2[s] = rdma(acc_a.at[ds(s * sub, sub)],
                         recv_a2.at[ds(s * sub, sub)], 4 + s, r)
            a2[s].start()
            b1[s].wait()
            acc_b[ds(s * sub, sub), :] = (
                keep_b[ds(s * sub, sub), :] + recv_b1[ds(s * sub, sub), :])
            b2[s] = rdma(acc_b.at[ds(s * sub, sub)],
                         recv_b2.at[ds(s * sub, sub)], 6 + s, q)
            b2[s].start()

        a3 = [None, None]
        b3 = [None, None]
        for s in range(2):
            a2[s].wait()
            out_ref[ds(j * qtr + s * sub, sub), :] = (
                acc_a[ds(s * sub, sub), :] + recv_a2[ds(s * sub, sub), :])
            a3[s] = rdma(out_ref.at[ds(j * qtr + s * sub, sub)],
                         out_ref.at[ds(j * qtr + s * sub, sub)], 8 + s, q)
            a3[s].start()
            b2[s].wait()
            out_ref[ds(half + jb * qtr + s * sub, sub), :] = (
                acc_b[ds(s * sub, sub), :] + recv_b2[ds(s * sub, sub), :])
            b3[s] = rdma(out_ref.at[ds(half + jb * qtr + s * sub, sub)],
                         out_ref.at[ds(half + jb * qtr + s * sub, sub)],
                         10 + s, r)
            b3[s].start()

        for s in range(2):
            a3[s].wait()
            b3[s].wait()

    return pl.pallas_call(
        body,
        out_shape=jax.ShapeDtypeStruct((m, n), jnp.bfloat16),
        in_specs=[pl.BlockSpec(memory_space=pltpu.VMEM)],
        out_specs=pl.BlockSpec(memory_space=pltpu.VMEM),
        scratch_shapes=[
            pltpu.VMEM((qtr, n), jnp.bfloat16),
            pltpu.VMEM((qtr, n), jnp.bfloat16),
            pltpu.VMEM((qtr, n), jnp.bfloat16),
            pltpu.VMEM((qtr, n), jnp.bfloat16),
            pltpu.VMEM((qtr, n), jnp.bfloat16),
            pltpu.VMEM((qtr, n), jnp.bfloat16),
            pltpu.VMEM((qtr, n), jnp.bfloat16),
            pltpu.VMEM((qtr, n), jnp.bfloat16),
            pltpu.VMEM((qtr, n), jnp.bfloat16),
            pltpu.VMEM((qtr, n), jnp.bfloat16),
            pltpu.SemaphoreType.DMA((12,)),
            pltpu.SemaphoreType.DMA((12,)),
        ],
        compiler_params=pltpu.CompilerParams(collective_id=0),
    )(x)
